# baseline (speedup 1.0000x reference)
import functools

import jax
import jax.numpy as jnp
from jax import lax
from jax.experimental import pallas as pl
from jax.experimental.pallas import tpu as pltpu
from jax.experimental.pallas import tpu_sc as plsc

_NWORDS = 1000000
_EMB = 64
_NIDX = 16384 * 50

_NC, _NS, _NL = 2, 16, 16
_NW = _NC * _NS
_BPW = _NIDX // _NW

_GRP = _EMB
_VCOLS = _EMB * _GRP
_VROWS = _NWORDS * _EMB // _VCOLS
_BR = 512
_NBLK = (_VROWS + _BR - 1) // _BR


def _rowsum_body(x_ref, m_ref, o_ref):
    o_ref[...] = jax.lax.dot(
        x_ref[...],
        m_ref[...],
        precision=jax.lax.Precision.HIGHEST,
        preferred_element_type=jnp.float32,
    )


_rowsum_call = pl.pallas_call(
    _rowsum_body,
    grid=(_NBLK,),
    in_specs=[
        pl.BlockSpec((_BR, _VCOLS), lambda i: (i, 0)),
        pl.BlockSpec((_VCOLS, _GRP), lambda i: (0, 0)),
    ],
    out_specs=pl.BlockSpec((_BR, _GRP), lambda i: (i, 0)),
    out_shape=jax.ShapeDtypeStruct((_VROWS, _GRP), jnp.float32),
)


@functools.partial(
    pl.kernel,
    mesh=plsc.VectorSubcoreMesh(core_axis_name="c", subcore_axis_name="s"),
    out_type=jax.ShapeDtypeStruct((_NW, _NL), jnp.float32),
    scratch_types=[
        pltpu.VMEM((_BPW,), jnp.int32),
        pltpu.VMEM((_BPW,), jnp.float32),
        pltpu.VMEM((_NL,), jnp.float32),
        pltpu.SemaphoreType.DMA,
    ],
)
def _sc_gather_sum(words_hbm, rowsums_hbm, out_hbm, idx_v, vals_v, acc_v, sem):
    wid = lax.axis_index("s") * _NC + lax.axis_index("c")
    base = wid * _BPW
    pltpu.sync_copy(words_hbm.at[pl.ds(base, _BPW)], idx_v)
    pltpu.async_copy(rowsums_hbm.at[idx_v], vals_v, sem).wait()

    zero = jnp.zeros((_NL,), jnp.float32)

    def body(i, accs):
        a0, a1, a2, a3 = accs
        b = i * 4 * _NL
        a0 = a0 + vals_v[pl.ds(b, _NL)]
        a1 = a1 + vals_v[pl.ds(b + _NL, _NL)]
        a2 = a2 + vals_v[pl.ds(b + 2 * _NL, _NL)]
        a3 = a3 + vals_v[pl.ds(b + 3 * _NL, _NL)]
        return (a0, a1, a2, a3)

    a0, a1, a2, a3 = lax.fori_loop(
        0, _BPW // (4 * _NL), body, (zero, zero, zero, zero)
    )
    acc_v[...] = (a0 + a1) + (a2 + a3)
    pltpu.sync_copy(acc_v, out_hbm.at[wid])


def _mlp_body(p_ref, w1_ref, b1_ref, w2_ref, b2_ref, wo_ref, bo_ref, o_ref):
    s = jnp.sum(p_ref[...])
    h1 = jnp.tanh(s * w1_ref[...] + b1_ref[...])
    h2 = jnp.tanh(
        jnp.dot(h1, w2_ref[...], preferred_element_type=jnp.float32) + b2_ref[...]
    )
    o_ref[...] = (
        jnp.dot(h2, wo_ref[...], preferred_element_type=jnp.float32) + bo_ref[...]
    )


def _mlp_call(partials, W1, b1, W2, b2, Wout, bout):
    return pl.pallas_call(
        _mlp_body,
        out_shape=jax.ShapeDtypeStruct((1, bout.shape[-1]), jnp.float32),
    )(partials, W1, b1, W2, b2, Wout, bout)


def kernel(words, table, W1, b1, W2, b2, Wout, bout):
    words_flat = words.reshape(-1).astype(jnp.int32)
    table_v = table.reshape(_VROWS, _VCOLS)
    fold = jnp.repeat(jnp.eye(_GRP, dtype=jnp.float32), _GRP, axis=0)
    rowsums = _rowsum_call(table_v, fold).reshape(-1)
    partials = _sc_gather_sum(words_flat, rowsums)
    return _mlp_call(
        partials,
        W1,
        b1.reshape(1, -1),
        W2,
        b2.reshape(1, -1),
        Wout,
        bout.reshape(1, -1),
    )


# speedup vs baseline: 1.0047x; 1.0047x over previous
import functools

import jax
import jax.numpy as jnp
from jax import lax
from jax.experimental import pallas as pl
from jax.experimental.pallas import tpu as pltpu
from jax.experimental.pallas import tpu_sc as plsc

_NWORDS = 1000000
_EMB = 64
_NIDX = 16384 * 50

_NC, _NS, _NL = 2, 16, 16
_NW = _NC * _NS
_BPW = _NIDX // _NW

_VROWS = _NWORDS // 2
_BR = 8192
_NBLK = (_VROWS + _BR - 1) // _BR
_OBR = 2 * _BR // 128


def _rowsum_body(x_ref, m_ref, o_ref):
    y = jax.lax.dot(
        x_ref[...],
        m_ref[...],
        precision=jax.lax.Precision.HIGHEST,
        preferred_element_type=jnp.float32,
    )
    o_ref[...] = jnp.concatenate(
        [y[128 * j : 128 * (j + 1), :] for j in range(_BR // 128)], axis=1
    )


_rowsum_call = pl.pallas_call(
    _rowsum_body,
    grid=(_NBLK,),
    in_specs=[
        pl.BlockSpec((_BR, 128), lambda i: (i, 0)),
        pl.BlockSpec((128, 2), lambda i: (0, 0)),
    ],
    out_specs=pl.BlockSpec((_OBR, 128), lambda i: (i, 0)),
    out_shape=jax.ShapeDtypeStruct((_NBLK * _OBR, 128), jnp.float32),
)


@functools.partial(
    pl.kernel,
    mesh=plsc.VectorSubcoreMesh(core_axis_name="c", subcore_axis_name="s"),
    out_type=jax.ShapeDtypeStruct((_NW, _NL), jnp.float32),
    scratch_types=[
        pltpu.VMEM((_BPW,), jnp.int32),
        pltpu.VMEM((_BPW,), jnp.float32),
        pltpu.VMEM((_NL,), jnp.float32),
        pltpu.SemaphoreType.DMA,
    ],
)
def _sc_gather_sum(words_hbm, rowsums_hbm, out_hbm, idx_v, vals_v, acc_v, sem):
    wid = lax.axis_index("s") * _NC + lax.axis_index("c")
    base = wid * _BPW
    pltpu.sync_copy(words_hbm.at[pl.ds(base, _BPW)], idx_v)

    def remap(i, _):
        v = idx_v[pl.ds(i * _NL, _NL)]
        r = v & 16383
        t = r >> 1
        p = (v - r) + ((t & 127) << 7) + ((t >> 7) << 1) + (v & 1)
        idx_v[pl.ds(i * _NL, _NL)] = p
        return 0

    lax.fori_loop(0, _BPW // _NL, remap, 0)
    pltpu.async_copy(rowsums_hbm.at[idx_v], vals_v, sem).wait()

    zero = jnp.zeros((_NL,), jnp.float32)

    def body(i, accs):
        a0, a1, a2, a3 = accs
        b = i * 4 * _NL
        a0 = a0 + vals_v[pl.ds(b, _NL)]
        a1 = a1 + vals_v[pl.ds(b + _NL, _NL)]
        a2 = a2 + vals_v[pl.ds(b + 2 * _NL, _NL)]
        a3 = a3 + vals_v[pl.ds(b + 3 * _NL, _NL)]
        return (a0, a1, a2, a3)

    a0, a1, a2, a3 = lax.fori_loop(
        0, _BPW // (4 * _NL), body, (zero, zero, zero, zero)
    )
    acc_v[...] = (a0 + a1) + (a2 + a3)
    pltpu.sync_copy(acc_v, out_hbm.at[wid])


def _mlp_body(p_ref, w1_ref, b1_ref, w2_ref, b2_ref, wo_ref, bo_ref, o_ref):
    s = jnp.sum(p_ref[...])
    h1 = jnp.tanh(s * w1_ref[...] + b1_ref[...])
    h2 = jnp.tanh(
        jnp.dot(h1, w2_ref[...], preferred_element_type=jnp.float32) + b2_ref[...]
    )
    o_ref[...] = (
        jnp.dot(h2, wo_ref[...], preferred_element_type=jnp.float32) + bo_ref[...]
    )


def _mlp_call(partials, W1, b1, W2, b2, Wout, bout):
    return pl.pallas_call(
        _mlp_body,
        out_shape=jax.ShapeDtypeStruct((1, bout.shape[-1]), jnp.float32),
    )(partials, W1, b1, W2, b2, Wout, bout)


def kernel(words, table, W1, b1, W2, b2, Wout, bout):
    words_flat = words.reshape(-1).astype(jnp.int32)
    table_v = table.reshape(_VROWS, 128)
    fold = jnp.repeat(jnp.eye(2, dtype=jnp.float32), 64, axis=0)
    rowsums = _rowsum_call(table_v, fold).reshape(-1)
    partials = _sc_gather_sum(words_flat, rowsums)
    return _mlp_call(
        partials,
        W1,
        b1.reshape(1, -1),
        W2,
        b2.reshape(1, -1),
        Wout,
        bout.reshape(1, -1),
    )


# speedup vs baseline: 1.0060x; 1.0013x over previous
import functools

import jax
import jax.numpy as jnp
from jax import lax
from jax.experimental import pallas as pl
from jax.experimental.pallas import tpu as pltpu
from jax.experimental.pallas import tpu_sc as plsc

_NWORDS = 1000000
_EMB = 64
_NIDX = 16384 * 50

_NC, _NS, _NL = 2, 16, 16
_NW = _NC * _NS
_BPW = _NIDX // _NW

_VROWS = _NWORDS // 2
_BR = 8192
_NBLK = (_VROWS + _BR - 1) // _BR
_OBR = 2 * _BR // 128


def _rowsum_body(x_ref, m_ref, o_ref):
    x = x_ref[...].reshape(_BR, 128)
    y = jax.lax.dot(
        x,
        m_ref[...],
        precision=jax.lax.Precision.HIGHEST,
        preferred_element_type=jnp.float32,
    )
    o_ref[...] = jnp.concatenate(
        [y[128 * j : 128 * (j + 1), :] for j in range(_BR // 128)], axis=1
    )


_rowsum_call = pl.pallas_call(
    _rowsum_body,
    grid=(_NBLK,),
    in_specs=[
        pl.BlockSpec((_BR * 128,), lambda i: (i,)),
        pl.BlockSpec((128, 2), lambda i: (0, 0)),
    ],
    out_specs=pl.BlockSpec((_OBR, 128), lambda i: (i, 0)),
    out_shape=jax.ShapeDtypeStruct((_NBLK * _OBR, 128), jnp.float32),
)


@functools.partial(
    pl.kernel,
    mesh=plsc.VectorSubcoreMesh(core_axis_name="c", subcore_axis_name="s"),
    out_type=jax.ShapeDtypeStruct((_NW, _NL), jnp.float32),
    scratch_types=[
        pltpu.VMEM((_BPW,), jnp.int32),
        pltpu.VMEM((_BPW,), jnp.float32),
        pltpu.VMEM((_NL,), jnp.float32),
        pltpu.SemaphoreType.DMA,
    ],
)
def _sc_gather_sum(words_hbm, rowsums_hbm, out_hbm, idx_v, vals_v, acc_v, sem):
    wid = lax.axis_index("s") * _NC + lax.axis_index("c")
    base = wid * _BPW
    pltpu.sync_copy(words_hbm.at[pl.ds(base, _BPW)], idx_v)

    def remap(i, _):
        v = idx_v[pl.ds(i * _NL, _NL)]
        r = v & 16383
        t = r >> 1
        p = (v - r) + ((t & 127) << 7) + ((t >> 7) << 1) + (v & 1)
        idx_v[pl.ds(i * _NL, _NL)] = p
        return 0

    lax.fori_loop(0, _BPW // _NL, remap, 0)
    pltpu.async_copy(rowsums_hbm.at[idx_v], vals_v, sem).wait()

    zero = jnp.zeros((_NL,), jnp.float32)

    def body(i, accs):
        a0, a1, a2, a3 = accs
        b = i * 4 * _NL
        a0 = a0 + vals_v[pl.ds(b, _NL)]
        a1 = a1 + vals_v[pl.ds(b + _NL, _NL)]
        a2 = a2 + vals_v[pl.ds(b + 2 * _NL, _NL)]
        a3 = a3 + vals_v[pl.ds(b + 3 * _NL, _NL)]
        return (a0, a1, a2, a3)

    a0, a1, a2, a3 = lax.fori_loop(
        0, _BPW // (4 * _NL), body, (zero, zero, zero, zero)
    )
    acc_v[...] = (a0 + a1) + (a2 + a3)
    pltpu.sync_copy(acc_v, out_hbm.at[wid])


def _mlp_body(p_ref, w1_ref, b1_ref, w2_ref, b2_ref, wo_ref, bo_ref, o_ref):
    s = jnp.sum(p_ref[...])
    h1 = jnp.tanh(s * w1_ref[...] + b1_ref[...])
    h2 = jnp.tanh(
        jnp.dot(h1, w2_ref[...], preferred_element_type=jnp.float32) + b2_ref[...]
    )
    o_ref[...] = (
        jnp.dot(h2, wo_ref[...], preferred_element_type=jnp.float32) + bo_ref[...]
    )


def _mlp_call(partials, W1, b1, W2, b2, Wout, bout):
    return pl.pallas_call(
        _mlp_body,
        out_shape=jax.ShapeDtypeStruct((1, bout.shape[-1]), jnp.float32),
    )(partials, W1, b1, W2, b2, Wout, bout)


def kernel(words, table, W1, b1, W2, b2, Wout, bout):
    words_flat = words.reshape(-1).astype(jnp.int32)
    table_v = table.reshape(-1)
    fold = jnp.repeat(jnp.eye(2, dtype=jnp.float32), 64, axis=0)
    rowsums = _rowsum_call(table_v, fold).reshape(-1)
    partials = _sc_gather_sum(words_flat, rowsums)
    return _mlp_call(
        partials,
        W1,
        b1.reshape(1, -1),
        W2,
        b2.reshape(1, -1),
        Wout,
        bout.reshape(1, -1),
    )


# speedup vs baseline: 6.3720x; 6.3340x over previous
import functools

import jax
import jax.numpy as jnp
from jax import lax
from jax.experimental import pallas as pl
from jax.experimental.pallas import tpu as pltpu
from jax.experimental.pallas import tpu_sc as plsc

_NWORDS = 1000000
_EMB = 64
_NIDX = 16384 * 50

_NC, _NS, _NL = 2, 16, 16
_NW = _NC * _NS
_BPW = _NIDX // _NW

_BC = 16384
_NBLK = (_NWORDS + _BC - 1) // _BC


def _rowsum_body(x_ref, o_ref):
    o_ref[...] = jnp.sum(x_ref[...], axis=0)


_rowsum_call = pl.pallas_call(
    _rowsum_body,
    grid=(_NBLK,),
    in_specs=[pl.BlockSpec((_EMB, _BC), lambda i: (0, i))],
    out_specs=pl.BlockSpec((_BC,), lambda i: (i,)),
    out_shape=jax.ShapeDtypeStruct((_NWORDS,), jnp.float32),
)


@functools.partial(
    pl.kernel,
    mesh=plsc.VectorSubcoreMesh(core_axis_name="c", subcore_axis_name="s"),
    out_type=jax.ShapeDtypeStruct((_NW, _NL), jnp.float32),
    scratch_types=[
        pltpu.VMEM((_BPW,), jnp.int32),
        pltpu.VMEM((_BPW,), jnp.float32),
        pltpu.VMEM((_NL,), jnp.float32),
        pltpu.SemaphoreType.DMA,
    ],
)
def _sc_gather_sum(words_hbm, rowsums_hbm, out_hbm, idx_v, vals_v, acc_v, sem):
    wid = lax.axis_index("s") * _NC + lax.axis_index("c")
    base = wid * _BPW
    pltpu.sync_copy(words_hbm.at[pl.ds(base, _BPW)], idx_v)
    pltpu.async_copy(rowsums_hbm.at[idx_v], vals_v, sem).wait()

    zero = jnp.zeros((_NL,), jnp.float32)

    def body(i, accs):
        a0, a1, a2, a3 = accs
        b = i * 4 * _NL
        a0 = a0 + vals_v[pl.ds(b, _NL)]
        a1 = a1 + vals_v[pl.ds(b + _NL, _NL)]
        a2 = a2 + vals_v[pl.ds(b + 2 * _NL, _NL)]
        a3 = a3 + vals_v[pl.ds(b + 3 * _NL, _NL)]
        return (a0, a1, a2, a3)

    a0, a1, a2, a3 = lax.fori_loop(
        0, _BPW // (4 * _NL), body, (zero, zero, zero, zero)
    )
    acc_v[...] = (a0 + a1) + (a2 + a3)
    pltpu.sync_copy(acc_v, out_hbm.at[wid])


def _mlp_body(p_ref, w1_ref, b1_ref, w2_ref, b2_ref, wo_ref, bo_ref, o_ref):
    s = jnp.sum(p_ref[...])
    h1 = jnp.tanh(s * w1_ref[...] + b1_ref[...])
    h2 = jnp.tanh(
        jnp.dot(h1, w2_ref[...], preferred_element_type=jnp.float32) + b2_ref[...]
    )
    o_ref[...] = (
        jnp.dot(h2, wo_ref[...], preferred_element_type=jnp.float32) + bo_ref[...]
    )


def _mlp_call(partials, W1, b1, W2, b2, Wout, bout):
    return pl.pallas_call(
        _mlp_body,
        out_shape=jax.ShapeDtypeStruct((1, bout.shape[-1]), jnp.float32),
    )(partials, W1, b1, W2, b2, Wout, bout)


def kernel(words, table, W1, b1, W2, b2, Wout, bout):
    words_flat = words.T.reshape(-1).astype(jnp.int32)
    rowsums = _rowsum_call(table.T)
    partials = _sc_gather_sum(words_flat, rowsums)
    return _mlp_call(
        partials,
        W1,
        b1.reshape(1, -1),
        W2,
        b2.reshape(1, -1),
        Wout,
        bout.reshape(1, -1),
    )
